# baseline (speedup 1.0000x reference)
import jax
import jax.numpy as jnp
from jax import lax
from jax.experimental import pallas as pl
from jax.experimental.pallas import tpu as pltpu
from jax.experimental.pallas import tpu_sc as plsc

VOCAB = 76
SEQ = 2048
DMODEL = 512
BATCH = 64

NCORES = 2
NSUB = 16
NW = NCORES * NSUB
TPS = SEQ // NW
HB = TPS // 2
NLANE = 16


def _pe_body(o_ref):
    r = lax.broadcasted_iota(jnp.int32, (SEQ, DMODEL), 0).astype(jnp.float32)
    c = lax.broadcasted_iota(jnp.int32, (SEQ, DMODEL), 1)
    even = (c - lax.rem(c, 2)).astype(jnp.float32)
    denom = jnp.exp(even * (jnp.log(10000.0) / DMODEL))
    theta = r / denom
    o_ref[...] = jnp.where(lax.rem(c, 2) == 0, jnp.sin(theta), jnp.cos(theta))


_pe_table = pl.pallas_call(
    _pe_body,
    out_shape=jax.ShapeDtypeStruct((SEQ, DMODEL), jnp.float32),
)


def _sc_body(idx_hbm, table_hbm, pe_hbm, out_hbm,
             pe_v, idx_v, gb0, gb1, ob0, ob1, gsem0, gsem1, osem0, osem1):
    cid = lax.axis_index("c")
    sid = lax.axis_index("s")
    wid = sid * NCORES + cid
    pltpu.sync_copy(pe_hbm.at[wid], pe_v)
    pltpu.sync_copy(idx_hbm.at[wid], idx_v)

    gbufs = ((gb0, gsem0), (gb1, gsem1))
    obufs = ((ob0, osem0), (ob1, osem1))

    def gsrc(b, h):
        return table_hbm.at[idx_v.at[b, pl.ds(h * HB, HB)]]

    def out_dst(b, h):
        return out_hbm.at[b, wid, pl.ds(h * HB, HB)]

    def compute(h, gb, ob):
        def r_body(r, carry):
            for j in range(DMODEL // NLANE):
                sl = pl.ds(j * NLANE, NLANE)
                ob[r, sl] = gb[r, sl] + pe_v[h * HB + r, sl]
            return carry

        lax.fori_loop(0, HB, r_body, 0)

    pltpu.async_copy(gsrc(0, 0), gb0, gsem0)

    def b_body(b, carry):
        for h in range(2):
            gb, gsem = gbufs[h]
            ob, osem = obufs[h]

            if h == 0:
                pltpu.async_copy(gsrc(b, 1), gb1, gsem1)
            else:
                @pl.when(b + 1 < BATCH)
                def _pref():
                    pltpu.async_copy(gsrc(b + 1, 0), gb0, gsem0)

            pltpu.make_async_copy(gsrc(b, h), gb, gsem).wait()

            @pl.when(b > 0)
            def _wait_out():
                pltpu.make_async_copy(ob, out_dst(b - 1, h), osem).wait()

            compute(h, gb, ob)
            pltpu.async_copy(ob, out_dst(b, h), osem)
        return carry

    lax.fori_loop(0, BATCH, b_body, 0)

    for h, (ob, osem) in enumerate(obufs):
        pltpu.make_async_copy(ob, out_dst(BATCH - 1, h), osem).wait()


_sc_embed = pl.kernel(
    _sc_body,
    out_type=jax.ShapeDtypeStruct((BATCH, NW, TPS, DMODEL), jnp.float32),
    mesh=plsc.VectorSubcoreMesh(core_axis_name="c", subcore_axis_name="s",
                                num_cores=NCORES, num_subcores=NSUB),
    scratch_types=[
        pltpu.VMEM((TPS, DMODEL), jnp.float32),
        pltpu.VMEM((BATCH, TPS), jnp.int32),
        pltpu.VMEM((HB, DMODEL), jnp.float32),
        pltpu.VMEM((HB, DMODEL), jnp.float32),
        pltpu.VMEM((HB, DMODEL), jnp.float32),
        pltpu.VMEM((HB, DMODEL), jnp.float32),
        pltpu.SemaphoreType.DMA,
        pltpu.SemaphoreType.DMA,
        pltpu.SemaphoreType.DMA,
        pltpu.SemaphoreType.DMA,
    ],
)


def kernel(x, embedding):
    idx = x.astype(jnp.int32).reshape(BATCH, NW, TPS).transpose(1, 0, 2)
    vstride = VOCAB + 5
    idx = idx + (jnp.arange(NW, dtype=jnp.int32) * vstride)[:, None, None]
    table_rep = jnp.zeros((NW, vstride, DMODEL), jnp.float32)
    table_rep = table_rep.at[:, :VOCAB].set(embedding[None])
    table_rep = table_rep.reshape(NW * vstride, DMODEL)
    pe = _pe_table().reshape(NW, TPS, DMODEL)
    out = _sc_embed(idx, table_rep, pe)
    return out.reshape(BATCH, SEQ, DMODEL)


# speedup vs baseline: 1.0433x; 1.0433x over previous
import jax
import jax.numpy as jnp
from jax import lax
from jax.experimental import pallas as pl
from jax.experimental.pallas import tpu as pltpu
from jax.experimental.pallas import tpu_sc as plsc

VOCAB = 76
SEQ = 2048
DMODEL = 512
BATCH = 64

NCORES = 2
NSUB = 16
NW = NCORES * NSUB
TPS = SEQ // NW
HB = TPS // 2
NLANE = 16


def _pe_body(o_ref):
    r = lax.broadcasted_iota(jnp.int32, (SEQ, DMODEL), 0).astype(jnp.float32)
    c = lax.broadcasted_iota(jnp.int32, (SEQ, DMODEL), 1)
    even = (c - lax.rem(c, 2)).astype(jnp.float32)
    denom = jnp.exp(even * (jnp.log(10000.0) / DMODEL))
    theta = r / denom
    o_ref[...] = jnp.where(lax.rem(c, 2) == 0, jnp.sin(theta), jnp.cos(theta))


_pe_table = pl.pallas_call(
    _pe_body,
    out_shape=jax.ShapeDtypeStruct((SEQ, DMODEL), jnp.float32),
)


def _sc_body(idx_hbm, table_hbm, pe_hbm, out_hbm,
             pe_v, idx_v, gb0, gb1, ob0, ob1,
             gsem0a, gsem0b, gsem1a, gsem1b, osem0, osem1):
    cid = lax.axis_index("c")
    sid = lax.axis_index("s")
    wid = sid * NCORES + cid
    pltpu.sync_copy(pe_hbm.at[wid], pe_v)
    pltpu.sync_copy(idx_hbm.at[wid], idx_v)

    HH = HB // 2
    gbufs = ((gb0, gsem0a, gsem0b), (gb1, gsem1a, gsem1b))
    obufs = ((ob0, osem0), (ob1, osem1))

    def gsrc(b, h, p):
        return table_hbm.at[idx_v.at[b, pl.ds(h * HB + p * HH, HH)]]

    def gather(b, h, gbuf):
        gb, sa, sb = gbuf
        pltpu.async_copy(gsrc(b, h, 0), gb.at[pl.ds(0, HH)], sa)
        pltpu.async_copy(gsrc(b, h, 1), gb.at[pl.ds(HH, HH)], sb)

    def gather_wait(b, h, gbuf):
        gb, sa, sb = gbuf
        pltpu.make_async_copy(gsrc(b, h, 0), gb.at[pl.ds(0, HH)], sa).wait()
        pltpu.make_async_copy(gsrc(b, h, 1), gb.at[pl.ds(HH, HH)], sb).wait()

    def out_dst(b, h):
        return out_hbm.at[b, wid, pl.ds(h * HB, HB)]

    def compute(h, gb, ob):
        def r_body(r, carry):
            for j in range(DMODEL // NLANE):
                sl = pl.ds(j * NLANE, NLANE)
                ob[r, sl] = gb[r, sl] + pe_v[h * HB + r, sl]
            return carry

        lax.fori_loop(0, HB, r_body, 0)

    gather(0, 0, gbufs[0])

    def b_body(b, carry):
        for h in range(2):
            gb = gbufs[h][0]
            ob, osem = obufs[h]

            if h == 0:
                gather(b, 1, gbufs[1])
            else:
                @pl.when(b + 1 < BATCH)
                def _pref():
                    gather(b + 1, 0, gbufs[0])

            gather_wait(b, h, gbufs[h])

            @pl.when(b > 0)
            def _wait_out():
                pltpu.make_async_copy(ob, out_dst(b - 1, h), osem).wait()

            compute(h, gb, ob)
            pltpu.async_copy(ob, out_dst(b, h), osem)
        return carry

    lax.fori_loop(0, BATCH, b_body, 0)

    for h, (ob, osem) in enumerate(obufs):
        pltpu.make_async_copy(ob, out_dst(BATCH - 1, h), osem).wait()


_sc_embed = pl.kernel(
    _sc_body,
    out_type=jax.ShapeDtypeStruct((BATCH, NW, TPS, DMODEL), jnp.float32),
    mesh=plsc.VectorSubcoreMesh(core_axis_name="c", subcore_axis_name="s",
                                num_cores=NCORES, num_subcores=NSUB),
    scratch_types=[
        pltpu.VMEM((TPS, DMODEL), jnp.float32),
        pltpu.VMEM((BATCH, TPS), jnp.int32),
        pltpu.VMEM((HB, DMODEL), jnp.float32),
        pltpu.VMEM((HB, DMODEL), jnp.float32),
        pltpu.VMEM((HB, DMODEL), jnp.float32),
        pltpu.VMEM((HB, DMODEL), jnp.float32),
        pltpu.SemaphoreType.DMA,
        pltpu.SemaphoreType.DMA,
        pltpu.SemaphoreType.DMA,
        pltpu.SemaphoreType.DMA,
        pltpu.SemaphoreType.DMA,
        pltpu.SemaphoreType.DMA,
    ],
)


def kernel(x, embedding):
    idx = x.astype(jnp.int32).reshape(BATCH, NW, TPS).transpose(1, 0, 2)
    idx = idx + (jnp.arange(NW, dtype=jnp.int32) * VOCAB)[:, None, None]
    table_rep = jnp.broadcast_to(embedding[None], (NW, VOCAB, DMODEL))
    table_rep = table_rep.reshape(NW * VOCAB, DMODEL)
    pe = _pe_table().reshape(NW, TPS, DMODEL)
    out = _sc_embed(idx, table_rep, pe)
    return out.reshape(BATCH, SEQ, DMODEL)
